# baseline (speedup 1.0000x reference)
import functools

import jax
import jax.numpy as jnp
from jax import lax
from jax.experimental import pallas as pl
from jax.experimental.pallas import tpu as pltpu
from jax.experimental.pallas import tpu_sc as plsc


def _pack16(hi_f32, lo_f32):
    hb = jax.lax.bitcast_convert_type(hi_f32.astype(jnp.bfloat16),
                                      jnp.uint16).astype(jnp.uint32)
    lb = jax.lax.bitcast_convert_type(lo_f32.astype(jnp.bfloat16),
                                      jnp.uint16).astype(jnp.uint32)
    return jax.lax.bitcast_convert_type((hb << 16) | lb, jnp.int32)


def _unpack_hi(w_i32):
    return jax.lax.bitcast_convert_type(w_i32 & jnp.int32(-65536), jnp.float32)


def _unpack_lo(w_i32):
    return jax.lax.bitcast_convert_type(w_i32 << 16, jnp.float32)


def _linears_body(d, x_ref, wa, ba, wb, bb, wd, bd, we, be,
                  ax_ref, tpk_ref, ex_ref):
    x = x_ref[...]
    ax_ref[...] = jnp.dot(x, wa[...], preferred_element_type=jnp.float32) + ba[...]
    dx = jnp.dot(x, wd[...], preferred_element_type=jnp.float32) + bd[...]
    bx = jnp.dot(x, wb[...], preferred_element_type=jnp.float32) + bb[...]
    tpk_ref[...] = _pack16(dx, bx)
    ex_ref[...] = jnp.dot(x, we[...], preferred_element_type=jnp.float32) + be[...]


def _make_gather(size, cbase, d):
    mesh = plsc.VectorSubcoreMesh(core_axis_name="c", subcore_axis_name="s")
    nw = 32
    per_w = size // nw
    k = 80
    nblk = per_w // k
    assert nblk * k == per_w

    @functools.partial(
        pl.kernel, mesh=mesh,
        out_type=[jax.ShapeDtypeStruct((size, d), jnp.int32),
                  jax.ShapeDtypeStruct((size, d), jnp.float32)],
        scratch_types=[pltpu.VMEM((2, k), jnp.int32),
                       pltpu.VMEM((2, k), jnp.int32),
                       pltpu.VMEM((2, k, d), jnp.int32),
                       pltpu.VMEM((2, k, d), jnp.float32),
                       pltpu.SemaphoreType.DMA,
                       pltpu.SemaphoreType.DMA,
                       pltpu.SemaphoreType.DMA,
                       pltpu.SemaphoreType.DMA],
    )
    def gather_k(tpk_hbm, ex_hbm, src_hbm, dst_hbm, gpk_hbm, ge_hbm,
                 src_v, dst_v, gpk_v, ge_v, semi0, semi1, semg0, semg1):
        semis = (semi0, semi1)
        semgs = (semg0, semg1)
        wid = lax.axis_index("s") * 2 + lax.axis_index("c")
        base0 = wid * per_w

        def issue_idx(j, b):
            base = cbase + base0 + j * k
            pltpu.async_copy(src_hbm.at[pl.ds(base, k)], src_v.at[b],
                             semis[b])
            pltpu.async_copy(dst_hbm.at[pl.ds(base, k)], dst_v.at[b],
                             semis[b])

        def wait_idx(b):
            pltpu.make_async_copy(src_hbm.at[pl.ds(0, k)], src_v.at[b],
                                  semis[b]).wait()
            pltpu.make_async_copy(dst_hbm.at[pl.ds(0, k)], dst_v.at[b],
                                  semis[b]).wait()

        def issue_gather(b):
            pltpu.async_copy(tpk_hbm.at[src_v.at[b]], gpk_v.at[b], semgs[b])
            pltpu.async_copy(ex_hbm.at[dst_v.at[b]], ge_v.at[b], semgs[b])

        def wait_gather(b):
            pltpu.make_async_copy(tpk_hbm.at[src_v.at[b]], gpk_v.at[b],
                                  semgs[b]).wait()
            pltpu.make_async_copy(ex_hbm.at[dst_v.at[b]], ge_v.at[b],
                                  semgs[b]).wait()

        def writeback(j, b):
            base = base0 + j * k
            pltpu.sync_copy(gpk_v.at[b], gpk_hbm.at[pl.ds(base, k)])
            pltpu.sync_copy(ge_v.at[b], ge_hbm.at[pl.ds(base, k)])

        issue_idx(0, 0)
        issue_idx(1, 1)
        wait_idx(0)
        issue_gather(0)

        @pl.loop(0, nblk // 2)
        def _(m):
            j0 = 2 * m
            wait_idx(1)
            wait_gather(0)
            issue_gather(1)
            writeback(j0, 0)

            @pl.when(j0 + 2 < nblk)
            def _():
                issue_idx(j0 + 2, 0)

            wait_gather(1)
            writeback(j0 + 1, 1)

            @pl.when(j0 + 3 < nblk)
            def _():
                issue_idx(j0 + 3, 1)

            @pl.when(j0 + 2 < nblk)
            def _():
                wait_idx(0)
                issue_gather(0)

        if nblk % 2 == 1:
            wait_gather(0)
            writeback(nblk - 1, 0)

    return gather_k


def _edge_body(d, nsteps, has_prev, *refs):
    if has_prev:
        (ex_blk, gpk_blk, ge_blk, sn_blk, wc, bc, _pm, _ps, _py,
         msg_ref, sig_ref, y_ref, stats_ref) = refs
    else:
        (ex_blk, gpk_blk, ge_blk, sn_blk, wc, bc,
         msg_ref, sig_ref, y_ref, stats_ref) = refs
    i = pl.program_id(0)
    ce = jnp.dot(ex_blk[...], wc[...], preferred_element_type=jnp.float32) + bc[...]
    gpk = gpk_blk[...]
    e = ce + _unpack_hi(gpk) + ge_blk[...]
    sig = jax.nn.sigmoid(e)
    msg_ref[...] = sig * _unpack_lo(gpk)
    sig_ref[...] = sig
    y = e * sn_blk[...]
    y_ref[...] = y

    @pl.when(i == 0)
    def _():
        stats_ref[...] = jnp.zeros_like(stats_ref)

    stats_ref[0:1, :] += jnp.sum(y, axis=0, keepdims=True)
    stats_ref[1:2, :] += jnp.sum(y * y, axis=0, keepdims=True)


def _make_segsum(ne, n2, d):
    mesh = plsc.VectorSubcoreMesh(core_axis_name="c", subcore_axis_name="s")
    ns = 16
    per_t = ne // ns
    k = 80
    nblk = per_t // k
    half = n2 // 2
    stripe = half // ns

    @functools.partial(
        pl.kernel, mesh=mesh,
        out_type=jax.ShapeDtypeStruct((n2, d), jnp.float32),
        scratch_types=[pltpu.VMEM((2, k), jnp.int32),
                       pltpu.VMEM((2, k, d), jnp.float32),
                       pltpu.VMEM((stripe, d), jnp.float32),
                       pltpu.VMEM_SHARED((half + 8, d), jnp.float32),
                       pltpu.SemaphoreType.DMA,
                       pltpu.SemaphoreType.DMA,
                       pltpu.SemaphoreType.DMA,
                       pltpu.SemaphoreType.DMA],
    )
    def segsum_k(vals_hbm, dst_hbm, z_hbm, out_hbm,
                 idx_v, vals_v, zbuf, acc, semi0, semi1, semv0, semv1):
        semis = (semi0, semi1)
        semvs = (semv0, semv1)
        cid = lax.axis_index("c")
        sid = lax.axis_index("s")
        base_node = cid * half

        r0 = sid * stripe
        pltpu.sync_copy(z_hbm, zbuf)
        pltpu.sync_copy(zbuf, acc.at[pl.ds(r0, stripe)])

        base0 = sid * per_t

        def issue(j, b):
            base = base0 + j * k
            pltpu.async_copy(dst_hbm.at[pl.ds(base, k)], idx_v.at[b], semis[b])
            pltpu.async_copy(vals_hbm.at[pl.ds(base, k)], vals_v.at[b],
                             semvs[b])

        def drain_transform_scatter(b):
            pltpu.make_async_copy(dst_hbm.at[pl.ds(0, k)],
                                  idx_v.at[b], semis[b]).wait()
            pltpu.make_async_copy(vals_hbm.at[pl.ds(0, k)],
                                  vals_v.at[b], semvs[b]).wait()

            @pl.loop(0, k // 16)
            def _(c):
                v = idx_v[b, pl.ds(c * 16, 16)] - base_node
                ok = (v >= 0) & (v < half)
                idx_v[b, pl.ds(c * 16, 16)] = jnp.where(ok, v, half)

            pltpu.sync_copy(vals_v.at[b], acc.at[idx_v.at[b]], add=True)

        plsc.subcore_barrier()

        issue(0, 0)

        @pl.loop(0, nblk // 2)
        def _(m):
            j0 = 2 * m
            issue(j0 + 1, 1)
            drain_transform_scatter(0)

            @pl.when(j0 + 2 < nblk)
            def _():
                issue(j0 + 2, 0)

            drain_transform_scatter(1)

        plsc.subcore_barrier()

        pltpu.sync_copy(acc.at[pl.ds(r0, stripe)], zbuf)
        pltpu.sync_copy(zbuf, out_hbm.at[pl.ds(base_node + r0, stripe)])

    return segsum_k


def _h_body(x_ref, ax_ref, num_ref, den_ref, sn_ref, g_ref, b_ref,
            h_ref):
    x = x_ref[...]
    den = den_ref[...]
    safe_den = jnp.where(den != 0.0, den, 1.0)
    h = jnp.where(den > 0.0, ax_ref[...] + num_ref[...] / safe_den, x)
    h = h * sn_ref[...]
    mu = jnp.mean(h, axis=0, keepdims=True)
    var = jnp.mean((h - mu) ** 2, axis=0, keepdims=True)
    hn = (h - mu) / jnp.sqrt(var + 1e-5) * g_ref[...] + b_ref[...]
    h_ref[...] = x + jnp.maximum(hn, 0.0)


def _e_body(ne, y_ref, exin_ref, s0_ref, s1_ref, g_ref, b_ref, e_ref):
    stats = s0_ref[...] + s1_ref[...]
    inv = 1.0 / ne
    mu = stats[0:1, :] * inv
    var = stats[1:2, :] * inv - mu * mu
    yn = (y_ref[...] - mu) / jnp.sqrt(var + 1e-5) * g_ref[...] + b_ref[...]
    e_ref[...] = exin_ref[...] + jnp.maximum(yn, 0.0)


def kernel(X, E_X, edge_index, snorm_n, snorm_e, Wa, ba, Wb, bb, Wc, bc,
           Wd, bd, We, be, gamma_h, beta_h, gamma_e, beta_e):
    n, d = X.shape
    ne = E_X.shape[0]

    ba2 = ba.reshape(1, d)
    bb2 = bb.reshape(1, d)
    bc2 = bc.reshape(1, d)
    bd2 = bd.reshape(1, d)
    be2 = be.reshape(1, d)
    gh2 = gamma_h.reshape(1, d)
    bh2 = beta_h.reshape(1, d)
    ge2 = gamma_e.reshape(1, d)
    bte2 = beta_e.reshape(1, d)

    src = edge_index[0].astype(jnp.int32)
    dst = edge_index[1].astype(jnp.int32)

    ax, tpk, exd = pl.pallas_call(
        functools.partial(_linears_body, d),
        out_shape=[jax.ShapeDtypeStruct((n, d), jnp.float32),
                   jax.ShapeDtypeStruct((n, d), jnp.int32),
                   jax.ShapeDtypeStruct((n, d), jnp.float32)],
    )(X, Wa, ba2, Wb, bb2, Wd, bd2, We, be2)

    blk = 2560
    nch = 2
    q = ne * 2 // 3 // 2560 * 2560
    csize = [q, ne - q]
    cbase = [0, q]

    msg = sig = y = None
    stats_list = []
    for ci in range(nch):
        size, base = csize[ci], cbase[ci]
        gpk_c, ge_c = _make_gather(size, base, d)(tpk, exd, src, dst)
        nsteps = size // blk
        bb = base // blk
        has_prev = ci > 0
        om = lambda i, b=bb: (i + b, 0)
        in_specs = [pl.BlockSpec((blk, d), om),
                    pl.BlockSpec((blk, d), lambda i: (i, 0)),
                    pl.BlockSpec((blk, d), lambda i: (i, 0)),
                    pl.BlockSpec((blk, 1), om),
                    pl.BlockSpec((d, d), lambda i: (0, 0)),
                    pl.BlockSpec((1, d), lambda i: (0, 0))]
        args = [E_X, gpk_c, ge_c, snorm_e, Wc, bc2]
        aliases = {}
        if has_prev:
            in_specs += [pl.BlockSpec(memory_space=pl.ANY)] * 3
            args += [msg, sig, y]
            aliases = {6: 0, 7: 1, 8: 2}
        msg, sig, y, stats_c = pl.pallas_call(
            functools.partial(_edge_body, d, nsteps, has_prev),
            grid=(nsteps,),
            in_specs=in_specs,
            out_specs=[pl.BlockSpec((blk, d), om),
                       pl.BlockSpec((blk, d), om),
                       pl.BlockSpec((blk, d), om),
                       pl.BlockSpec((8, d), lambda i: (0, 0))],
            out_shape=[jax.ShapeDtypeStruct((ne, d), jnp.float32),
                       jax.ShapeDtypeStruct((ne, d), jnp.float32),
                       jax.ShapeDtypeStruct((ne, d), jnp.float32),
                       jax.ShapeDtypeStruct((8, d), jnp.float32)],
            input_output_aliases=aliases,
            compiler_params=pltpu.CompilerParams(
                dimension_semantics=("arbitrary",)),
        )(*args)
        stats_list.append(stats_c)

    n2 = ((n + 255) // 256) * 256
    zs = jnp.zeros((n2 // 32, d), jnp.float32)
    segsum = _make_segsum(ne, n2, d)
    num = segsum(msg, dst, zs)[:n]
    den = segsum(sig, dst, zs)[:n]

    H = pl.pallas_call(
        _h_body,
        out_shape=jax.ShapeDtypeStruct((n, d), jnp.float32),
    )(X, ax, num, den, snorm_n, gh2, bh2)

    E = pl.pallas_call(
        functools.partial(_e_body, float(ne)),
        grid=(ne // blk,),
        in_specs=[pl.BlockSpec((blk, d), lambda i: (i, 0)),
                  pl.BlockSpec((blk, d), lambda i: (i, 0)),
                  pl.BlockSpec((8, d), lambda i: (0, 0)),
                  pl.BlockSpec((8, d), lambda i: (0, 0)),
                  pl.BlockSpec((1, d), lambda i: (0, 0)),
                  pl.BlockSpec((1, d), lambda i: (0, 0))],
        out_specs=pl.BlockSpec((blk, d), lambda i: (i, 0)),
        out_shape=jax.ShapeDtypeStruct((ne, d), jnp.float32),
        compiler_params=pltpu.CompilerParams(
            dimension_semantics=("arbitrary",)),
    )(y, E_X, stats_list[0], stats_list[1], ge2, bte2)

    return (H, E)


# speedup vs baseline: 1.0386x; 1.0386x over previous
import functools

import jax
import jax.numpy as jnp
from jax import lax
from jax.experimental import pallas as pl
from jax.experimental.pallas import tpu as pltpu
from jax.experimental.pallas import tpu_sc as plsc


def _pack16(hi_f32, lo_f32):
    hb = jax.lax.bitcast_convert_type(hi_f32.astype(jnp.bfloat16),
                                      jnp.uint16).astype(jnp.uint32)
    lb = jax.lax.bitcast_convert_type(lo_f32.astype(jnp.bfloat16),
                                      jnp.uint16).astype(jnp.uint32)
    return jax.lax.bitcast_convert_type((hb << 16) | lb, jnp.int32)


def _unpack_hi(w_i32):
    return jax.lax.bitcast_convert_type(w_i32 & jnp.int32(-65536), jnp.float32)


def _unpack_lo(w_i32):
    return jax.lax.bitcast_convert_type(w_i32 << 16, jnp.float32)


def _linears_body(d, x_ref, wa, ba, wb, bb, wd, bd, we, be,
                  ax_ref, tpk_ref, ex_ref):
    x = x_ref[...]
    ax_ref[...] = jnp.dot(x, wa[...], preferred_element_type=jnp.float32) + ba[...]
    dx = jnp.dot(x, wd[...], preferred_element_type=jnp.float32) + bd[...]
    bx = jnp.dot(x, wb[...], preferred_element_type=jnp.float32) + bb[...]
    tpk_ref[...] = _pack16(dx, bx)
    ex_ref[...] = jnp.dot(x, we[...], preferred_element_type=jnp.float32) + be[...]


def _make_gather(size, cbase, d):
    mesh = plsc.VectorSubcoreMesh(core_axis_name="c", subcore_axis_name="s")
    nw = 32
    per_w = size // nw
    k = 80
    nblk = per_w // k
    assert nblk * k == per_w

    @functools.partial(
        pl.kernel, mesh=mesh,
        out_type=[jax.ShapeDtypeStruct((size, d), jnp.int32),
                  jax.ShapeDtypeStruct((size, d), jnp.float32)],
        scratch_types=[pltpu.VMEM((2, k), jnp.int32),
                       pltpu.VMEM((2, k), jnp.int32),
                       pltpu.VMEM((2, k, d), jnp.int32),
                       pltpu.VMEM((2, k, d), jnp.float32),
                       pltpu.SemaphoreType.DMA,
                       pltpu.SemaphoreType.DMA,
                       pltpu.SemaphoreType.DMA,
                       pltpu.SemaphoreType.DMA],
    )
    def gather_k(tpk_hbm, ex_hbm, src_hbm, dst_hbm, gpk_hbm, ge_hbm,
                 src_v, dst_v, gpk_v, ge_v, semi0, semi1, semg0, semg1):
        semis = (semi0, semi1)
        semgs = (semg0, semg1)
        wid = lax.axis_index("s") * 2 + lax.axis_index("c")
        base0 = wid * per_w

        def issue_idx(j, b):
            base = cbase + base0 + j * k
            pltpu.async_copy(src_hbm.at[pl.ds(base, k)], src_v.at[b],
                             semis[b])
            pltpu.async_copy(dst_hbm.at[pl.ds(base, k)], dst_v.at[b],
                             semis[b])

        def wait_idx(b):
            pltpu.make_async_copy(src_hbm.at[pl.ds(0, k)], src_v.at[b],
                                  semis[b]).wait()
            pltpu.make_async_copy(dst_hbm.at[pl.ds(0, k)], dst_v.at[b],
                                  semis[b]).wait()

        def issue_gather(b):
            pltpu.async_copy(tpk_hbm.at[src_v.at[b]], gpk_v.at[b], semgs[b])
            pltpu.async_copy(ex_hbm.at[dst_v.at[b]], ge_v.at[b], semgs[b])

        def wait_gather(b):
            pltpu.make_async_copy(tpk_hbm.at[src_v.at[b]], gpk_v.at[b],
                                  semgs[b]).wait()
            pltpu.make_async_copy(ex_hbm.at[dst_v.at[b]], ge_v.at[b],
                                  semgs[b]).wait()

        def writeback(j, b):
            base = base0 + j * k
            pltpu.sync_copy(gpk_v.at[b], gpk_hbm.at[pl.ds(base, k)])
            pltpu.sync_copy(ge_v.at[b], ge_hbm.at[pl.ds(base, k)])

        issue_idx(0, 0)
        issue_idx(1, 1)
        wait_idx(0)
        issue_gather(0)

        @pl.loop(0, nblk // 2)
        def _(m):
            j0 = 2 * m
            wait_idx(1)
            wait_gather(0)
            issue_gather(1)
            writeback(j0, 0)

            @pl.when(j0 + 2 < nblk)
            def _():
                issue_idx(j0 + 2, 0)

            wait_gather(1)
            writeback(j0 + 1, 1)

            @pl.when(j0 + 3 < nblk)
            def _():
                issue_idx(j0 + 3, 1)

            @pl.when(j0 + 2 < nblk)
            def _():
                wait_idx(0)
                issue_gather(0)

        if nblk % 2 == 1:
            wait_gather(0)
            writeback(nblk - 1, 0)

    return gather_k


def _edge_body(d, nsteps, has_prev, *refs):
    if has_prev:
        (ex_blk, gpk_blk, ge_blk, sn_blk, wc, bc, _pm, _ps, _py,
         msg_ref, sig_ref, y_ref, stats_ref) = refs
    else:
        (ex_blk, gpk_blk, ge_blk, sn_blk, wc, bc,
         msg_ref, sig_ref, y_ref, stats_ref) = refs
    i = pl.program_id(0)
    ce = jnp.dot(ex_blk[...], wc[...], preferred_element_type=jnp.float32) + bc[...]
    gpk = gpk_blk[...]
    e = ce + _unpack_hi(gpk) + ge_blk[...]
    sig = jax.nn.sigmoid(e)
    msg_ref[...] = sig * _unpack_lo(gpk)
    sig_ref[...] = sig
    y = e * sn_blk[...]
    y_ref[...] = y.astype(jnp.bfloat16)

    @pl.when(i == 0)
    def _():
        stats_ref[...] = jnp.zeros_like(stats_ref)

    stats_ref[0:1, :] += jnp.sum(y, axis=0, keepdims=True)
    stats_ref[1:2, :] += jnp.sum(y * y, axis=0, keepdims=True)


def _make_segsum(ne, n2, d):
    mesh = plsc.VectorSubcoreMesh(core_axis_name="c", subcore_axis_name="s")
    ns = 16
    per_t = ne // ns
    k = 80
    nblk = per_t // k
    half = n2 // 2
    stripe = half // ns

    @functools.partial(
        pl.kernel, mesh=mesh,
        out_type=jax.ShapeDtypeStruct((n2, d), jnp.float32),
        scratch_types=[pltpu.VMEM((2, k), jnp.int32),
                       pltpu.VMEM((2, k, d), jnp.float32),
                       pltpu.VMEM((stripe, d), jnp.float32),
                       pltpu.VMEM_SHARED((half + 8, d), jnp.float32),
                       pltpu.SemaphoreType.DMA,
                       pltpu.SemaphoreType.DMA,
                       pltpu.SemaphoreType.DMA,
                       pltpu.SemaphoreType.DMA],
    )
    def segsum_k(vals_hbm, dst_hbm, z_hbm, out_hbm,
                 idx_v, vals_v, zbuf, acc, semi0, semi1, semv0, semv1):
        semis = (semi0, semi1)
        semvs = (semv0, semv1)
        cid = lax.axis_index("c")
        sid = lax.axis_index("s")
        base_node = cid * half

        r0 = sid * stripe
        pltpu.sync_copy(z_hbm, zbuf)
        pltpu.sync_copy(zbuf, acc.at[pl.ds(r0, stripe)])

        base0 = sid * per_t

        def issue(j, b):
            base = base0 + j * k
            pltpu.async_copy(dst_hbm.at[pl.ds(base, k)], idx_v.at[b], semis[b])
            pltpu.async_copy(vals_hbm.at[pl.ds(base, k)], vals_v.at[b],
                             semvs[b])

        def drain_transform_scatter(b):
            pltpu.make_async_copy(dst_hbm.at[pl.ds(0, k)],
                                  idx_v.at[b], semis[b]).wait()
            pltpu.make_async_copy(vals_hbm.at[pl.ds(0, k)],
                                  vals_v.at[b], semvs[b]).wait()

            @pl.loop(0, k // 16)
            def _(c):
                v = idx_v[b, pl.ds(c * 16, 16)] - base_node
                ok = (v >= 0) & (v < half)
                idx_v[b, pl.ds(c * 16, 16)] = jnp.where(ok, v, half)

            pltpu.sync_copy(vals_v.at[b], acc.at[idx_v.at[b]], add=True)

        plsc.subcore_barrier()

        issue(0, 0)

        @pl.loop(0, nblk // 2)
        def _(m):
            j0 = 2 * m
            issue(j0 + 1, 1)
            drain_transform_scatter(0)

            @pl.when(j0 + 2 < nblk)
            def _():
                issue(j0 + 2, 0)

            drain_transform_scatter(1)

        plsc.subcore_barrier()

        pltpu.sync_copy(acc.at[pl.ds(r0, stripe)], zbuf)
        pltpu.sync_copy(zbuf, out_hbm.at[pl.ds(base_node + r0, stripe)])

    return segsum_k


def _h_body(x_ref, ax_ref, num_ref, den_ref, sn_ref, g_ref, b_ref,
            h_ref):
    x = x_ref[...]
    den = den_ref[...]
    safe_den = jnp.where(den != 0.0, den, 1.0)
    h = jnp.where(den > 0.0, ax_ref[...] + num_ref[...] / safe_den, x)
    h = h * sn_ref[...]
    mu = jnp.mean(h, axis=0, keepdims=True)
    var = jnp.mean((h - mu) ** 2, axis=0, keepdims=True)
    hn = (h - mu) / jnp.sqrt(var + 1e-5) * g_ref[...] + b_ref[...]
    h_ref[...] = x + jnp.maximum(hn, 0.0)


def _e_body(ne, y_ref, exin_ref, s0_ref, s1_ref, s2_ref, s3_ref,
            g_ref, b_ref, e_ref):
    stats = (s0_ref[...] + s1_ref[...]) + (s2_ref[...] + s3_ref[...])
    inv = 1.0 / ne
    mu = stats[0:1, :] * inv
    var = stats[1:2, :] * inv - mu * mu
    yv = y_ref[...].astype(jnp.float32)
    yn = (yv - mu) / jnp.sqrt(var + 1e-5) * g_ref[...] + b_ref[...]
    e_ref[...] = exin_ref[...] + jnp.maximum(yn, 0.0)


def kernel(X, E_X, edge_index, snorm_n, snorm_e, Wa, ba, Wb, bb, Wc, bc,
           Wd, bd, We, be, gamma_h, beta_h, gamma_e, beta_e):
    n, d = X.shape
    ne = E_X.shape[0]

    ba2 = ba.reshape(1, d)
    bb2 = bb.reshape(1, d)
    bc2 = bc.reshape(1, d)
    bd2 = bd.reshape(1, d)
    be2 = be.reshape(1, d)
    gh2 = gamma_h.reshape(1, d)
    bh2 = beta_h.reshape(1, d)
    ge2 = gamma_e.reshape(1, d)
    bte2 = beta_e.reshape(1, d)

    src = edge_index[0].astype(jnp.int32)
    dst = edge_index[1].astype(jnp.int32)

    ax, tpk, exd = pl.pallas_call(
        functools.partial(_linears_body, d),
        out_shape=[jax.ShapeDtypeStruct((n, d), jnp.float32),
                   jax.ShapeDtypeStruct((n, d), jnp.int32),
                   jax.ShapeDtypeStruct((n, d), jnp.float32)],
    )(X, Wa, ba2, Wb, bb2, Wd, bd2, We, be2)

    blk = 2560
    nch = 4
    q = ne // 4 // 2560 * 2560
    csize = [q, q, q, ne - 3 * q]
    cbase = [0, q, 2 * q, 3 * q]

    msg = sig = y = None
    stats_list = []
    for ci in range(nch):
        size, base = csize[ci], cbase[ci]
        gpk_c, ge_c = _make_gather(size, base, d)(tpk, exd, src, dst)
        nsteps = size // blk
        bb = base // blk
        has_prev = ci > 0
        om = lambda i, b=bb: (i + b, 0)
        in_specs = [pl.BlockSpec((blk, d), om),
                    pl.BlockSpec((blk, d), lambda i: (i, 0)),
                    pl.BlockSpec((blk, d), lambda i: (i, 0)),
                    pl.BlockSpec((blk, 1), om),
                    pl.BlockSpec((d, d), lambda i: (0, 0)),
                    pl.BlockSpec((1, d), lambda i: (0, 0))]
        args = [E_X, gpk_c, ge_c, snorm_e, Wc, bc2]
        aliases = {}
        if has_prev:
            in_specs += [pl.BlockSpec(memory_space=pl.ANY)] * 3
            args += [msg, sig, y]
            aliases = {6: 0, 7: 1, 8: 2}
        msg, sig, y, stats_c = pl.pallas_call(
            functools.partial(_edge_body, d, nsteps, has_prev),
            grid=(nsteps,),
            in_specs=in_specs,
            out_specs=[pl.BlockSpec((blk, d), om),
                       pl.BlockSpec((blk, d), om),
                       pl.BlockSpec((blk, d), om),
                       pl.BlockSpec((8, d), lambda i: (0, 0))],
            out_shape=[jax.ShapeDtypeStruct((ne, d), jnp.float32),
                       jax.ShapeDtypeStruct((ne, d), jnp.float32),
                       jax.ShapeDtypeStruct((ne, d), jnp.bfloat16),
                       jax.ShapeDtypeStruct((8, d), jnp.float32)],
            input_output_aliases=aliases,
            compiler_params=pltpu.CompilerParams(
                dimension_semantics=("arbitrary",)),
        )(*args)
        stats_list.append(stats_c)

    n2 = ((n + 255) // 256) * 256
    zs = jnp.zeros((n2 // 32, d), jnp.float32)
    segsum = _make_segsum(ne, n2, d)
    num = segsum(msg, dst, zs)[:n]
    den = segsum(sig, dst, zs)[:n]

    H = pl.pallas_call(
        _h_body,
        out_shape=jax.ShapeDtypeStruct((n, d), jnp.float32),
    )(X, ax, num, den, snorm_n, gh2, bh2)

    E = pl.pallas_call(
        functools.partial(_e_body, float(ne)),
        grid=(ne // blk,),
        in_specs=[pl.BlockSpec((blk, d), lambda i: (i, 0)),
                  pl.BlockSpec((blk, d), lambda i: (i, 0)),
                  pl.BlockSpec((8, d), lambda i: (0, 0)),
                  pl.BlockSpec((8, d), lambda i: (0, 0)),
                  pl.BlockSpec((8, d), lambda i: (0, 0)),
                  pl.BlockSpec((8, d), lambda i: (0, 0)),
                  pl.BlockSpec((1, d), lambda i: (0, 0)),
                  pl.BlockSpec((1, d), lambda i: (0, 0))],
        out_specs=pl.BlockSpec((blk, d), lambda i: (i, 0)),
        out_shape=jax.ShapeDtypeStruct((ne, d), jnp.float32),
        compiler_params=pltpu.CompilerParams(
            dimension_semantics=("arbitrary",)),
    )(y, E_X, stats_list[0], stats_list[1], stats_list[2], stats_list[3],
      ge2, bte2)

    return (H, E)
